# baseline (speedup 1.0000x reference)
import functools
import math

import jax
import jax.numpy as jnp
from jax import lax
from jax.experimental import pallas as pl
from jax.experimental.pallas import tpu as pltpu
from jax.experimental.pallas import tpu_sc as plsc

_SCALE = 30.0
_MARGIN = 0.35
_C2 = _SCALE / math.log(2.0)
_NEG = float("-inf")

_B = 1024
_C = 100000
_BR = 32
_L = 16
_NC = 2
_NS = 16
_NW = _NC * _NS

_B_SC = 512
_B_TC = _B - _B_SC
_RPW = _B_SC // _NW

_SLAB = 8
_NSLAB = _RPW // _SLAB
_CWC = 4096
_NFULL = _C // _CWC
_NPAIR = _NFULL // 2
_TOFF = _NFULL * _CWC
_TAIL = _C - _TOFF
_NVC = _CWC // _L
_NVT = _TAIL // _L


def _rowstats_body(x_ref, tgt_ref, m_ref, s_ref, t_ref):
    x = x_ref[...]
    m = jnp.max(x, axis=1, keepdims=True)
    s = jnp.sum(jnp.exp2((x - m) * _C2), axis=1, keepdims=True)
    cols = lax.broadcasted_iota(jnp.int32, x.shape, 1)
    tv = jnp.max(jnp.where(cols == tgt_ref[...], x, _NEG), axis=1, keepdims=True)
    m_ref[...] = m
    s_ref[...] = s
    t_ref[...] = tv


def _rowstats_tc(output, tgt2d):
    stat = pl.BlockSpec((_BR, 1), lambda i: (i, 0))
    return pl.pallas_call(
        _rowstats_body,
        grid=(_B_TC // _BR,),
        in_specs=[pl.BlockSpec((_BR, _C), lambda i: (i, 0)), stat],
        out_specs=[stat, stat, stat],
        out_shape=[
            jax.ShapeDtypeStruct((_B_TC, 1), jnp.float32),
            jax.ShapeDtypeStruct((_B_TC, 1), jnp.float32),
            jax.ShapeDtypeStruct((_B_TC, 1), jnp.float32),
        ],
        compiler_params=pltpu.CompilerParams(
            dimension_semantics=("parallel",),
        ),
    )(output, tgt2d)


def _sc_rowstats_body(x_hbm, tgtb_hbm, m_hbm, s_hbm, t_hbm,
                      tb_v, buf0, buf1, buft, mres, sres, tres, sem, tsem):
    wid = lax.axis_index("s") * _NC + lax.axis_index("c")
    base = wid * _RPW
    lane = lax.iota(jnp.int32, _L)

    def slab_copy(row0, coff, cw, buf, s):
        return pltpu.make_async_copy(
            x_hbm.at[pl.ds(row0, _SLAB), pl.ds(coff, cw)], buf, s)

    def chunk_sweep(buf, nv, cw, col0, t_b, ms, ss, tvs):
        def maxbody(j, mm):
            return tuple(
                jnp.maximum(mm[r], buf[r, pl.ds(j * _L, _L)])
                for r in range(_SLAB)
            )

        m_new = list(lax.fori_loop(0, nv, maxbody, tuple(ms)))
        ss = [ss[r] * jnp.exp((ms[r] - m_new[r]) * _SCALE) for r in range(_SLAB)]

        def sumbody(j, carry):
            sacc, tvacc = carry
            cols = (col0 + j * _L) + lane
            ns, ntv = [], []
            for r in range(_SLAB):
                v = buf[r, pl.ds(j * _L, _L)]
                ns.append(sacc[r] + jnp.exp((v - m_new[r]) * _SCALE))
                ntv.append(jnp.maximum(tvacc[r], jnp.where(cols == t_b[r], v, _NEG)))
            return (tuple(ns), tuple(ntv))

        ss, tvs = lax.fori_loop(0, nv, sumbody, (tuple(ss), tuple(tvs)))
        return m_new, list(ss), list(tvs)

    for slab in range(_NSLAB):
        row0 = base + slab * _SLAB

        pltpu.async_copy(
            tgtb_hbm.at[pl.ds(row0, _SLAB), pl.ds(0, _L)], tb_v, tsem
        ).wait()
        t_b = [tb_v[r, pl.ds(0, _L)] for r in range(_SLAB)]

        m_a = [jnp.full((_L,), _NEG, jnp.float32) for _ in range(_SLAB)]
        s_a = [jnp.zeros((_L,), jnp.float32) for _ in range(_SLAB)]
        tv_a = [jnp.full((_L,), _NEG, jnp.float32) for _ in range(_SLAB)]

        slab_copy(row0, 0, _CWC, buf0, sem.at[0]).start()
        slab_copy(row0, _TOFF, _TAIL, buft, sem.at[2]).start()
        slab_copy(row0, _CWC, _CWC, buf1, sem.at[1]).start()

        def pair_body(p, carry):
            ms, ss, tvs = carry
            ms, ss, tvs = list(ms), list(ss), list(tvs)
            for b, buf in ((0, buf0), (1, buf1)):
                c = 2 * p + b
                col0 = c * _CWC
                slab_copy(row0, col0, _CWC, buf, sem.at[b]).wait()
                ms, ss, tvs = chunk_sweep(buf, _NVC, _CWC, col0, t_b, ms, ss, tvs)

                @pl.when(p + 1 < _NPAIR)
                def _():
                    slab_copy(row0, (2 * (p + 1) + b) * _CWC, _CWC, buf, sem.at[b]).start()

            return (tuple(ms), tuple(ss), tuple(tvs))

        m_a, s_a, tv_a = lax.fori_loop(
            0, _NPAIR, pair_body, (tuple(m_a), tuple(s_a), tuple(tv_a)))
        m_a, s_a, tv_a = list(m_a), list(s_a), list(tv_a)

        slab_copy(row0, _TOFF, _TAIL, buft, sem.at[2]).wait()
        m_a, s_a, tv_a = chunk_sweep(buft, _NVT, _TAIL, _TOFF, t_b, m_a, s_a, tv_a)

        for r8 in range(_SLAB):
            r = slab * _SLAB + r8
            mres[r, pl.ds(0, _L)] = m_a[r8]
            sres[r, pl.ds(0, _L)] = s_a[r8]
            tres[r, pl.ds(0, _L)] = tv_a[r8]

    pltpu.sync_copy(mres, m_hbm.at[pl.ds(base, _RPW)])
    pltpu.sync_copy(sres, s_hbm.at[pl.ds(base, _RPW)])
    pltpu.sync_copy(tres, t_hbm.at[pl.ds(base, _RPW)])


@functools.cache
def _sc_rowstats_call():
    part = jax.ShapeDtypeStruct((_B_SC, _L), jnp.float32)
    return pl.kernel(
        _sc_rowstats_body,
        out_type=(part, part, part),
        mesh=plsc.VectorSubcoreMesh(
            core_axis_name="c", subcore_axis_name="s", num_cores=_NC, num_subcores=_NS
        ),
        compiler_params=pltpu.CompilerParams(use_tc_tiling_on_sc=True),
        scratch_types=[
            pltpu.VMEM((_SLAB, _L), jnp.int32),
            pltpu.VMEM((_SLAB, _CWC), jnp.float32),
            pltpu.VMEM((_SLAB, _CWC), jnp.float32),
            pltpu.VMEM((_SLAB, _TAIL), jnp.float32),
            pltpu.VMEM((_RPW, _L), jnp.float32),
            pltpu.VMEM((_RPW, _L), jnp.float32),
            pltpu.VMEM((_RPW, _L), jnp.float32),
            pltpu.SemaphoreType.DMA((3,)),
            pltpu.SemaphoreType.DMA,
        ],
    )


def _nll_sum(m, s, tv):
    a = jnp.exp2((tv - m) * _C2)
    bb = jnp.exp2((tv - _MARGIN - m) * _C2)
    sp = s - a + bb
    return jnp.sum(_SCALE * m + jnp.log(sp) - _SCALE * (tv - _MARGIN))


def _combine_body(m1_ref, s1_ref, t1_ref, m2_ref, s2_ref, t2_ref, out_ref):
    m2p = m2_ref[...]
    m2 = jnp.max(m2p, axis=1, keepdims=True)
    s2 = jnp.sum(s2_ref[...] * jnp.exp2((m2p - m2) * _C2), axis=1, keepdims=True)
    t2 = jnp.max(t2_ref[...], axis=1, keepdims=True)
    tot = _nll_sum(m1_ref[...], s1_ref[...], t1_ref[...]) + _nll_sum(m2, s2, t2)
    out_ref[0, 0] = tot * (1.0 / _B)


def _combine(m1, s1, t1, m2, s2, t2):
    return pl.pallas_call(
        _combine_body,
        out_specs=pl.BlockSpec(memory_space=pltpu.SMEM),
        out_shape=jax.ShapeDtypeStruct((1, 1), jnp.float32),
    )(m1, s1, t1, m2, s2, t2)


def kernel(output, target):
    b, c = output.shape
    tgt = target.astype(jnp.int32)
    tgtb = jnp.broadcast_to(tgt[:, None], (b, _L))
    m2, s2, t2 = _sc_rowstats_call()(output[_B_TC:], tgtb[_B_TC:])
    m1, s1, t1 = _rowstats_tc(output, tgt.reshape(b, 1))
    loss = _combine(m1, s1, t1, m2, s2, t2)
    return loss[0, 0]


# speedup vs baseline: 3.9142x; 3.9142x over previous
import functools
import math

import jax
import jax.numpy as jnp
from jax import lax
from jax.experimental import pallas as pl
from jax.experimental.pallas import tpu as pltpu

_SCALE = 30.0
_MARGIN = 0.35
_C2 = _SCALE / math.log(2.0)
_NEG = float("-inf")

_B = 1024
_C = 100000
_CBR = 1000
_NSTEP = _C // _CBR


def _colstats_body(x_ref, tgt_ref, m_ref, s_ref, t_ref):
    i = pl.program_id(0)

    @pl.when(i == 0)
    def _():
        m_ref[...] = jnp.full((1, _B), _NEG, jnp.float32)
        s_ref[...] = jnp.zeros((1, _B), jnp.float32)
        t_ref[...] = jnp.full((1, _B), _NEG, jnp.float32)

    x = x_ref[...]
    bm = jnp.max(x, axis=0, keepdims=True)
    rows = i * _CBR + lax.broadcasted_iota(jnp.int32, x.shape, 0)
    btv = jnp.max(jnp.where(rows == tgt_ref[...], x, _NEG), axis=0, keepdims=True)

    m_old = m_ref[...]
    m_new = jnp.maximum(m_old, bm)
    s_ref[...] = s_ref[...] * jnp.exp2((m_old - m_new) * _C2) + jnp.sum(
        jnp.exp2((x - m_new) * _C2), axis=0, keepdims=True
    )
    m_ref[...] = m_new
    t_ref[...] = jnp.maximum(t_ref[...], btv)


def _colstats(xt, tgt1b):
    acc = pl.BlockSpec((1, _B), lambda i: (0, 0))
    return pl.pallas_call(
        _colstats_body,
        grid=(_NSTEP,),
        in_specs=[
            pl.BlockSpec((_CBR, _B), lambda i: (i, 0)),
            pl.BlockSpec((1, _B), lambda i: (0, 0)),
        ],
        out_specs=[acc, acc, acc],
        out_shape=[
            jax.ShapeDtypeStruct((1, _B), jnp.float32),
            jax.ShapeDtypeStruct((1, _B), jnp.float32),
            jax.ShapeDtypeStruct((1, _B), jnp.float32),
        ],
    )(xt, tgt1b)


def _combine_body(m_ref, s_ref, t_ref, out_ref):
    m = m_ref[...]
    s = s_ref[...]
    tv = t_ref[...]
    a = jnp.exp2((tv - m) * _C2)
    bb = jnp.exp2((tv - _MARGIN - m) * _C2)
    sp = s - a + bb
    nll = _SCALE * m + jnp.log(sp) - _SCALE * (tv - _MARGIN)
    out_ref[0, 0] = jnp.sum(nll) * (1.0 / _B)


def _combine(m, s, tv):
    return pl.pallas_call(
        _combine_body,
        out_specs=pl.BlockSpec(memory_space=pltpu.SMEM),
        out_shape=jax.ShapeDtypeStruct((1, 1), jnp.float32),
    )(m, s, tv)


def kernel(output, target):
    b, c = output.shape
    tgt = target.astype(jnp.int32)
    xt = output.T
    m, s, tv = _colstats(xt, tgt.reshape(1, b))
    loss = _combine(m, s, tv)
    return loss[0, 0]


# speedup vs baseline: 4.5994x; 1.1750x over previous
import functools
import math

import jax
import jax.numpy as jnp
from jax import lax
from jax.experimental import pallas as pl
from jax.experimental.pallas import tpu as pltpu

_SCALE = 30.0
_MARGIN = 0.35
_C2 = _SCALE / math.log(2.0)
_NEG = float("-inf")

_B = 1024
_C = 100000
_CBR = 2000
_NSTEP = _C // _CBR


def _colstats_body(x_ref, tgt_ref, m_ref, s_ref, t_ref):
    i = pl.program_id(0)

    @pl.when(i == 0)
    def _():
        m_ref[...] = jnp.full((1, _B), _NEG, jnp.float32)
        s_ref[...] = jnp.zeros((1, _B), jnp.float32)
        t_ref[...] = jnp.full((1, _B), _NEG, jnp.float32)

    x = x_ref[...]
    bm = jnp.max(x, axis=0, keepdims=True)
    rows = i * _CBR + lax.broadcasted_iota(jnp.int32, x.shape, 0)
    btv = jnp.max(jnp.where(rows == tgt_ref[...], x, _NEG), axis=0, keepdims=True)

    m_old = m_ref[...]
    m_new = jnp.maximum(m_old, bm)
    s_ref[...] = s_ref[...] * jnp.exp2((m_old - m_new) * _C2) + jnp.sum(
        jnp.exp2((x - m_new) * _C2), axis=0, keepdims=True
    )
    m_ref[...] = m_new
    t_ref[...] = jnp.maximum(t_ref[...], btv)


def _colstats(xt, tgt1b):
    acc = pl.BlockSpec((1, _B), lambda i: (0, 0))
    return pl.pallas_call(
        _colstats_body,
        grid=(_NSTEP,),
        in_specs=[
            pl.BlockSpec((_CBR, _B), lambda i: (i, 0)),
            pl.BlockSpec((1, _B), lambda i: (0, 0)),
        ],
        out_specs=[acc, acc, acc],
        out_shape=[
            jax.ShapeDtypeStruct((1, _B), jnp.float32),
            jax.ShapeDtypeStruct((1, _B), jnp.float32),
            jax.ShapeDtypeStruct((1, _B), jnp.float32),
        ],
    )(xt, tgt1b)


def _combine_body(m_ref, s_ref, t_ref, out_ref):
    m = m_ref[...]
    s = s_ref[...]
    tv = t_ref[...]
    a = jnp.exp2((tv - m) * _C2)
    bb = jnp.exp2((tv - _MARGIN - m) * _C2)
    sp = s - a + bb
    nll = _SCALE * m + jnp.log(sp) - _SCALE * (tv - _MARGIN)
    out_ref[0, 0] = jnp.sum(nll) * (1.0 / _B)


def _combine(m, s, tv):
    return pl.pallas_call(
        _combine_body,
        out_specs=pl.BlockSpec(memory_space=pltpu.SMEM),
        out_shape=jax.ShapeDtypeStruct((1, 1), jnp.float32),
    )(m, s, tv)


def kernel(output, target):
    b, c = output.shape
    tgt = target.astype(jnp.int32)
    xt = output.T
    m, s, tv = _colstats(xt, tgt.reshape(1, b))
    loss = _combine(m, s, tv)
    return loss[0, 0]


# speedup vs baseline: 4.8087x; 1.0455x over previous
import functools
import math

import jax
import jax.numpy as jnp
from jax import lax
from jax.experimental import pallas as pl
from jax.experimental.pallas import tpu as pltpu

_SCALE = 30.0
_MARGIN = 0.35
_C2 = _SCALE / math.log(2.0)
_NEG = float("-inf")

_B = 1024
_C = 100000
_CBR = 4000
_NSTEP = _C // _CBR


def _colstats_body(x_ref, tgt_ref, m_ref, s_ref, t_ref):
    i = pl.program_id(0)

    @pl.when(i == 0)
    def _():
        m_ref[...] = jnp.full((1, _B), _NEG, jnp.float32)
        s_ref[...] = jnp.zeros((1, _B), jnp.float32)
        t_ref[...] = jnp.full((1, _B), _NEG, jnp.float32)

    x = x_ref[...]
    bm = jnp.max(x, axis=0, keepdims=True)
    rows = i * _CBR + lax.broadcasted_iota(jnp.int32, x.shape, 0)
    btv = jnp.max(jnp.where(rows == tgt_ref[...], x, _NEG), axis=0, keepdims=True)

    m_old = m_ref[...]
    m_new = jnp.maximum(m_old, bm)
    s_ref[...] = s_ref[...] * jnp.exp2((m_old - m_new) * _C2) + jnp.sum(
        jnp.exp2((x - m_new) * _C2), axis=0, keepdims=True
    )
    m_ref[...] = m_new
    t_ref[...] = jnp.maximum(t_ref[...], btv)


def _colstats(xt, tgt1b):
    acc = pl.BlockSpec((1, _B), lambda i: (0, 0))
    return pl.pallas_call(
        _colstats_body,
        grid=(_NSTEP,),
        in_specs=[
            pl.BlockSpec((_CBR, _B), lambda i: (i, 0)),
            pl.BlockSpec((1, _B), lambda i: (0, 0)),
        ],
        out_specs=[acc, acc, acc],
        out_shape=[
            jax.ShapeDtypeStruct((1, _B), jnp.float32),
            jax.ShapeDtypeStruct((1, _B), jnp.float32),
            jax.ShapeDtypeStruct((1, _B), jnp.float32),
        ],
    )(xt, tgt1b)


def _combine_body(m_ref, s_ref, t_ref, out_ref):
    m = m_ref[...]
    s = s_ref[...]
    tv = t_ref[...]
    a = jnp.exp2((tv - m) * _C2)
    bb = jnp.exp2((tv - _MARGIN - m) * _C2)
    sp = s - a + bb
    nll = _SCALE * m + jnp.log(sp) - _SCALE * (tv - _MARGIN)
    out_ref[0, 0] = jnp.sum(nll) * (1.0 / _B)


def _combine(m, s, tv):
    return pl.pallas_call(
        _combine_body,
        out_specs=pl.BlockSpec(memory_space=pltpu.SMEM),
        out_shape=jax.ShapeDtypeStruct((1, 1), jnp.float32),
    )(m, s, tv)


def kernel(output, target):
    b, c = output.shape
    tgt = target.astype(jnp.int32)
    xt = output.T
    m, s, tv = _colstats(xt, tgt.reshape(1, b))
    loss = _combine(m, s, tv)
    return loss[0, 0]


# speedup vs baseline: 4.8694x; 1.0126x over previous
import functools
import math

import jax
import jax.numpy as jnp
from jax import lax
from jax.experimental import pallas as pl
from jax.experimental.pallas import tpu as pltpu

_SCALE = 30.0
_MARGIN = 0.35
_C2 = _SCALE / math.log(2.0)
_NEG = float("-inf")

_B = 1024
_C = 100000
_CBR = 5000
_NSTEP = _C // _CBR


def _colstats_body(x_ref, tgt_ref, m_ref, s_ref, t_ref):
    i = pl.program_id(0)

    @pl.when(i == 0)
    def _():
        m_ref[...] = jnp.full((1, _B), _NEG, jnp.float32)
        s_ref[...] = jnp.zeros((1, _B), jnp.float32)
        t_ref[...] = jnp.full((1, _B), _NEG, jnp.float32)

    x = x_ref[...]
    bm = jnp.max(x, axis=0, keepdims=True)
    rows = lax.broadcasted_iota(jnp.int32, x.shape, 0)
    tloc = tgt_ref[...] - i * _CBR
    btv = jnp.max(jnp.where(rows == tloc, x, _NEG), axis=0, keepdims=True)

    m_old = m_ref[...]
    m_new = jnp.maximum(m_old, bm)
    s_ref[...] = s_ref[...] * jnp.exp2((m_old - m_new) * _C2) + jnp.sum(
        jnp.exp2((x - m_new) * _C2), axis=0, keepdims=True
    )
    m_ref[...] = m_new
    t_ref[...] = jnp.maximum(t_ref[...], btv)


def _colstats(xt, tgt1b):
    acc = pl.BlockSpec((1, _B), lambda i: (0, 0))
    return pl.pallas_call(
        _colstats_body,
        grid=(_NSTEP,),
        in_specs=[
            pl.BlockSpec((_CBR, _B), lambda i: (i, 0)),
            pl.BlockSpec((1, _B), lambda i: (0, 0)),
        ],
        out_specs=[acc, acc, acc],
        out_shape=[
            jax.ShapeDtypeStruct((1, _B), jnp.float32),
            jax.ShapeDtypeStruct((1, _B), jnp.float32),
            jax.ShapeDtypeStruct((1, _B), jnp.float32),
        ],
    )(xt, tgt1b)


def _combine_body(m_ref, s_ref, t_ref, out_ref):
    m = m_ref[...]
    s = s_ref[...]
    tv = t_ref[...]
    a = jnp.exp2((tv - m) * _C2)
    bb = jnp.exp2((tv - _MARGIN - m) * _C2)
    sp = s - a + bb
    nll = _SCALE * m + jnp.log(sp) - _SCALE * (tv - _MARGIN)
    out_ref[0, 0] = jnp.sum(nll) * (1.0 / _B)


def _combine(m, s, tv):
    return pl.pallas_call(
        _combine_body,
        out_specs=pl.BlockSpec(memory_space=pltpu.SMEM),
        out_shape=jax.ShapeDtypeStruct((1, 1), jnp.float32),
    )(m, s, tv)


def kernel(output, target):
    b, c = output.shape
    tgt = target.astype(jnp.int32)
    xt = output.T
    m, s, tv = _colstats(xt, tgt.reshape(1, b))
    loss = _combine(m, s, tv)
    return loss[0, 0]
